# Initial kernel scaffold; baseline (speedup 1.0000x reference)
import jax
import jax.numpy as jnp
from jax.experimental import pallas as pl


def kernel(input_node_indices, node_features, edge_index, params):
    raise NotImplementedError("write your pallas kernel here")


# speedup vs baseline: 8.0666x; 8.0666x over previous
import jax
import jax.numpy as jnp
from jax import lax
from jax.experimental import pallas as pl
from jax.experimental.pallas import tpu as pltpu
from jax.experimental.pallas import tpu_sc as plsc

NC = 2
NS = 16
NW = NC * NS
G = 128
LW = 128


def _affine(p):
    scale = p["g"] / jnp.sqrt(p["v"] + 1e-3)
    shift = p["b"] - p["m"] * scale
    return scale[None, :], shift[None, :]


def _full(shape):
    return pl.BlockSpec(shape, lambda i: (0,) * len(shape))


def _msg_pad(m, blk, h):
    cols = lax.broadcasted_iota(jnp.int32, (blk, LW), 1)
    return jnp.where(cols == h, 1.0, m)


def _tc1_body(nf, s0, t0, W0, b0, s1, t1, W1, b1, x_out, m_out):
    x = jax.nn.gelu((nf[...] * s0[...] + t0[...]) @ W0[...] + b0[...])
    x_out[...] = x
    m = jax.nn.gelu((x * s1[...] + t1[...]) @ W1[...] + b1[...])
    m_out[...] = _msg_pad(m, x.shape[0], W0.shape[1])


def _tc1(nf, pre, prep, N, DF, H, BLK):
    s0, t0 = _affine(pre)
    s1, t1 = _affine(prep)
    W1p = jnp.zeros((H, LW), jnp.float32).at[:, :H].set(prep["W"])
    b1p = jnp.zeros((1, LW), jnp.float32).at[0, :H].set(prep["bias"])
    s1p = jnp.zeros((1, H), jnp.float32).at[0, :H].set(s1[0])
    return pl.pallas_call(
        _tc1_body,
        grid=(N // BLK,),
        in_specs=[
            pl.BlockSpec((BLK, DF), lambda i: (i, 0)),
            _full((1, DF)), _full((1, DF)), _full((DF, H)), _full((1, H)),
            _full((1, H)), _full((1, H)), _full((H, LW)), _full((1, LW)),
        ],
        out_specs=[pl.BlockSpec((BLK, H), lambda i: (i, 0)),
                   pl.BlockSpec((BLK, LW), lambda i: (i, 0))],
        out_shape=[jax.ShapeDtypeStruct((N, H), jnp.float32),
                   jax.ShapeDtypeStruct((N, LW), jnp.float32)],
    )(nf, s0, t0, pre["W"], pre["bias"][None, :], s1p, t1, W1p, b1p)


def _agg(p0, p1, h):
    s = p0[...] + p1[...]
    cnt = s[:, h:h + 1]
    return s[:, :h] / jnp.maximum(cnt, 1.0)


def _tc2_body(x, p0, p1, su, tu, Wu, bu, sp, tp, Wp, bp, x2_out, m2_out):
    H = x.shape[1]
    agg = _agg(p0, p1, H)
    h = jnp.concatenate([x[...], agg], axis=1)
    x2 = jax.nn.gelu((h * su[...] + tu[...]) @ Wu[...] + bu[...]) + x[...]
    x2_out[...] = x2
    m2 = jax.nn.gelu((x2 * sp[...] + tp[...]) @ Wp[...] + bp[...])
    m2_out[...] = _msg_pad(m2, x2.shape[0], H)


def _tc2(x, p0, p1, upd, prep, N, H, BLK):
    su, tu = _affine(upd)
    sp, tp = _affine(prep)
    Wpp = jnp.zeros((H, LW), jnp.float32).at[:, :H].set(prep["W"])
    bpp = jnp.zeros((1, LW), jnp.float32).at[0, :H].set(prep["bias"])
    blk_h = pl.BlockSpec((BLK, H), lambda i: (i, 0))
    blk_w = pl.BlockSpec((BLK, LW), lambda i: (i, 0))
    return pl.pallas_call(
        _tc2_body,
        grid=(N // BLK,),
        in_specs=[
            blk_h, blk_w, blk_w,
            _full((1, 2 * H)), _full((1, 2 * H)), _full((2 * H, H)), _full((1, H)),
            _full((1, H)), _full((1, H)), _full((H, LW)), _full((1, LW)),
        ],
        out_specs=[blk_h, blk_w],
        out_shape=[jax.ShapeDtypeStruct((N, H), jnp.float32),
                   jax.ShapeDtypeStruct((N, LW), jnp.float32)],
    )(x, p0, p1, su, tu, upd["W"], upd["bias"][None, :],
      sp, tp, Wpp, bpp)


def _tc3_body(x, p0, p1, su, tu, Wu, bu, so, to, Wo, bo, Wz, bz, z_out):
    H = x.shape[1]
    agg = _agg(p0, p1, H)
    h = jnp.concatenate([x[...], agg], axis=1)
    x3 = jax.nn.gelu((h * su[...] + tu[...]) @ Wu[...] + bu[...]) + x[...]
    y = jax.nn.gelu((x3 * so[...] + to[...]) @ Wo[...] + bo[...])
    z_out[...] = y @ Wz[...] + bz[...]


def _tc3(x, p0, p1, upd, post, Wz, bz, N, H, BLK):
    su, tu = _affine(upd)
    so, to = _affine(post)
    NCLS = Wz.shape[1]
    Wzp = jnp.zeros((H, LW), jnp.float32).at[:, :NCLS].set(Wz)
    bzp = jnp.zeros((1, LW), jnp.float32).at[0, :NCLS].set(bz)
    blk_h = pl.BlockSpec((BLK, H), lambda i: (i, 0))
    blk_w = pl.BlockSpec((BLK, LW), lambda i: (i, 0))
    return pl.pallas_call(
        _tc3_body,
        grid=(N // BLK,),
        in_specs=[
            blk_h, blk_w, blk_w,
            _full((1, 2 * H)), _full((1, 2 * H)), _full((2 * H, H)), _full((1, H)),
            _full((1, H)), _full((1, H)), _full((H, H)), _full((1, H)),
            _full((H, LW)), _full((1, LW)),
        ],
        out_specs=blk_w,
        out_shape=jax.ShapeDtypeStruct((N, LW), jnp.float32),
    )(x, p0, p1, su, tu, upd["W"], upd["bias"][None, :],
      so, to, post["W"], post["bias"][None, :], Wzp, bzp)


def _sc_conv(m, nbr2d, dst2d, zrows, K, NPAD, RT):
    mesh = plsc.VectorSubcoreMesh(core_axis_name="c", subcore_axis_name="s")

    def body(m_hbm, nbr_hbm, dst_hbm, z_hbm, s_out,
             idx_nbr, idx_dst, rows, acc, sem):
        c = lax.axis_index("c")
        s = lax.axis_index("s")
        wid = s * NC + c
        pltpu.sync_copy(nbr_hbm.at[pl.ds(wid * K, K)], idx_nbr)
        pltpu.sync_copy(dst_hbm.at[pl.ds(wid * K, K)], idx_dst)
        pltpu.sync_copy(z_hbm, acc.at[pl.ds(s * RT, RT)])
        plsc.subcore_barrier()

        def step(j, carry):
            pltpu.async_copy(m_hbm.at[idx_nbr.at[j]], rows, sem).wait()
            pltpu.sync_copy(rows, acc.at[idx_dst.at[j]], add=True)
            return carry

        lax.fori_loop(0, K, step, 0)
        plsc.subcore_barrier()
        pltpu.sync_copy(acc.at[pl.ds(s * RT, RT)],
                        s_out.at[c, pl.ds(s * RT, RT)])

    fn = pl.kernel(
        body,
        out_type=jax.ShapeDtypeStruct((NC, NPAD, LW), jnp.float32),
        mesh=mesh,
        scratch_types=[
            pltpu.VMEM((K, G), jnp.int32),
            pltpu.VMEM((K, G), jnp.int32),
            pltpu.VMEM((G, LW), jnp.float32),
            pltpu.VMEM_SHARED((NPAD, LW), jnp.float32),
            pltpu.SemaphoreType.DMA,
        ],
    )
    return fn(m, nbr2d, dst2d, zrows)


def _sc_gather(z, idx, B):
    bpw = B // NW
    mesh = plsc.VectorSubcoreMesh(core_axis_name="c", subcore_axis_name="s")

    def body(z_hbm, idx_hbm, out_hbm, idx_v, rows_v, sem):
        wid = lax.axis_index("s") * NC + lax.axis_index("c")
        base = wid * bpw
        pltpu.sync_copy(idx_hbm.at[pl.ds(base, bpw)], idx_v)
        pltpu.async_copy(z_hbm.at[idx_v], rows_v, sem).wait()
        pltpu.sync_copy(rows_v, out_hbm.at[pl.ds(base, bpw)])

    fn = pl.kernel(
        body,
        out_type=jax.ShapeDtypeStruct((B, LW), jnp.float32),
        mesh=mesh,
        scratch_types=[
            pltpu.VMEM((bpw,), jnp.int32),
            pltpu.VMEM((bpw, LW), jnp.float32),
            pltpu.SemaphoreType.DMA,
        ],
    )
    return fn(z, idx)


def kernel(input_node_indices, node_features, edge_index, params):
    N, DF = node_features.shape
    H = params["pre"]["W"].shape[1]
    NCLS = params["out_W"].shape[1]
    E = edge_index.shape[1]
    B = input_node_indices.shape[0]

    BLK = 1000
    K = (-(-E // (NW * G)) + 7) // 8 * 8
    EP = NW * K * G
    NPAD = ((N + 1 + NS * 8 - 1) // (NS * 8)) * (NS * 8)
    RT = NPAD // NS

    dst = edge_index[0]
    nbr = edge_index[1]
    pad = EP - E
    pad_ar = jnp.arange(pad, dtype=jnp.int32)
    nbr2d = jnp.concatenate([nbr, pad_ar % N]).reshape(NW * K, G)
    dst2d = jnp.concatenate([dst, N + pad_ar % (NPAD - N)]).reshape(NW * K, G)

    zrows = jnp.zeros((RT, LW), jnp.float32)

    p = params
    x, m1 = _tc1(node_features, p["pre"], p["c1_prep"], N, DF, H, BLK)
    s1 = _sc_conv(m1, nbr2d, dst2d, zrows, K, NPAD, RT)
    x2, m2 = _tc2(x, s1[0, :N], s1[1, :N], p["c1_upd"], p["c2_prep"], N, H, BLK)
    s2 = _sc_conv(m2, nbr2d, dst2d, zrows, K, NPAD, RT)
    z = _tc3(x2, s2[0, :N], s2[1, :N],
             p["c2_upd"], p["post"], p["out_W"], p["out_bias"], N, H, BLK)
    return _sc_gather(z, input_node_indices, B)[:, :NCLS]
